# baseline (speedup 1.0000x reference)
import functools

import jax
import jax.numpy as jnp
from jax import lax
from jax.experimental import pallas as pl
from jax.experimental.pallas import tpu as pltpu
from jax.experimental.pallas import tpu_sc as plsc

N = 10000
D = 128
E = 320000
NC, NS = 2, 16
NW = NC * NS
EPW = E // NW
CH = 80
NCHUNK = EPW // CH
RPT = N // NS

BN = 2000
GRID = N // BN


def _tc_transform_body(x_ref, w_ref, asrc_ref, adst_ref,
                       th_ref, as_ref, ad_ref):
    th = jnp.dot(x_ref[...], w_ref[...], preferred_element_type=jnp.float32)
    th_ref[...] = th
    as_ref[...] = jnp.sum(th * asrc_ref[...], axis=1, keepdims=True)
    ad_ref[...] = jnp.sum(th * adst_ref[...], axis=1, keepdims=True)


def _tc_transform(x, W, asrc, adst):
    return pl.pallas_call(
        _tc_transform_body,
        grid=(GRID,),
        in_specs=[
            pl.BlockSpec((BN, D), lambda i: (i, 0)),
            pl.BlockSpec((D, D), lambda i: (0, 0)),
            pl.BlockSpec((1, D), lambda i: (0, 0)),
            pl.BlockSpec((1, D), lambda i: (0, 0)),
        ],
        out_specs=[
            pl.BlockSpec((BN, D), lambda i: (i, 0)),
            pl.BlockSpec((BN, 1), lambda i: (i, 0)),
            pl.BlockSpec((BN, 1), lambda i: (i, 0)),
        ],
        out_shape=[
            jax.ShapeDtypeStruct((N, D), jnp.float32),
            jax.ShapeDtypeStruct((N, 1), jnp.float32),
            jax.ShapeDtypeStruct((N, 1), jnp.float32),
        ],
    )(x, W, asrc, adst)


def _tc_mid_body(n0_ref, n1_ref, d0_ref, d1_ref, b_ref, w_ref,
                 asrc_ref, adst_ref, th_ref, as_ref, ad_ref):
    den = d0_ref[0] + d1_ref[0] + 1e-16
    h = (n0_ref[0] + n1_ref[0]) / den + b_ref[...]
    h = jnp.maximum(h, 0.0)
    th = jnp.dot(h, w_ref[...], preferred_element_type=jnp.float32)
    th_ref[...] = th
    as_ref[...] = jnp.sum(th * asrc_ref[...], axis=1, keepdims=True)
    ad_ref[...] = jnp.sum(th * adst_ref[...], axis=1, keepdims=True)


def _tc_mid(num, den, b, W, asrc, adst):
    return pl.pallas_call(
        _tc_mid_body,
        grid=(GRID,),
        in_specs=[
            pl.BlockSpec((1, BN, D), lambda i: (0, i, 0)),
            pl.BlockSpec((1, BN, D), lambda i: (1, i, 0)),
            pl.BlockSpec((1, BN, 1), lambda i: (0, i, 0)),
            pl.BlockSpec((1, BN, 1), lambda i: (1, i, 0)),
            pl.BlockSpec((1, D), lambda i: (0, 0)),
            pl.BlockSpec((D, D), lambda i: (0, 0)),
            pl.BlockSpec((1, D), lambda i: (0, 0)),
            pl.BlockSpec((1, D), lambda i: (0, 0)),
        ],
        out_specs=[
            pl.BlockSpec((BN, D), lambda i: (i, 0)),
            pl.BlockSpec((BN, 1), lambda i: (i, 0)),
            pl.BlockSpec((BN, 1), lambda i: (i, 0)),
        ],
        out_shape=[
            jax.ShapeDtypeStruct((N, D), jnp.float32),
            jax.ShapeDtypeStruct((N, 1), jnp.float32),
            jax.ShapeDtypeStruct((N, 1), jnp.float32),
        ],
    )(num, num, den, den, b, W, asrc, adst)


def _tc_out_body(n0_ref, n1_ref, d0_ref, d1_ref, b_ref, out_ref):
    den = d0_ref[0] + d1_ref[0] + 1e-16
    out_ref[...] = (n0_ref[0] + n1_ref[0]) / den + b_ref[...]


def _tc_out(num, den, b):
    return pl.pallas_call(
        _tc_out_body,
        grid=(GRID,),
        in_specs=[
            pl.BlockSpec((1, BN, D), lambda i: (0, i, 0)),
            pl.BlockSpec((1, BN, D), lambda i: (1, i, 0)),
            pl.BlockSpec((1, BN, 1), lambda i: (0, i, 0)),
            pl.BlockSpec((1, BN, 1), lambda i: (1, i, 0)),
            pl.BlockSpec((1, D), lambda i: (0, 0)),
        ],
        out_specs=pl.BlockSpec((BN, D), lambda i: (i, 0)),
        out_shape=jax.ShapeDtypeStruct((N, D), jnp.float32),
    )(num, num, den, den, b)


def _sc_edge_body(theta_hbm, as_hbm, ad_hbm, pair_hbm, zeros_hbm,
                  rep_hbm, r80_hbm, num_hbm, den_hbm, acc_sp, den_sp,
                  pair_i, ase_v, ade_v, w_v, msg_v, den_v, rep_v, r80_v,
                  sem_rows, sem_as, sem_ad, sem_idx, sem_sc):
    c = lax.axis_index("c")
    s = lax.axis_index("s")
    w_id = c * NS + s
    pltpu.sync_copy(rep_hbm, rep_v)
    pltpu.sync_copy(r80_hbm, r80_v)

    def zden(i, carry):
        den_v[i // 8, pl.ds((i % 8) * 16, 16)] = jnp.zeros((16,),
                                                           jnp.float32)
        return carry
    lax.fori_loop(0, 80 * 8, zden, 0)
    pltpu.sync_copy(zeros_hbm.at[pl.ds(s * RPT, RPT)],
                    acc_sp.at[pl.ds(s * RPT, RPT)])
    pltpu.sync_copy(den_v.at[pl.ds(s * 5, 5)],
                    den_sp.at[pl.ds(s * 5, 5)])
    plsc.subcore_barrier()

    def start_idx(j):
        q = j % 4
        pltpu.async_copy(pair_hbm.at[w_id, j], pair_i.at[q], sem_idx.at[q])

    def wait_idx(j):
        q = j % 4
        pltpu.make_async_copy(pair_hbm.at[w_id, j], pair_i.at[q],
                              sem_idx.at[q]).wait()

    def start_gather(j):
        b, q = j % 3, j % 4
        pltpu.async_copy(theta_hbm.at[pair_i.at[q, 0]], msg_v.at[b],
                         sem_rows.at[b])
        pltpu.async_copy(as_hbm.at[pair_i.at[q, 0]], ase_v.at[b],
                         sem_as.at[b])
        pltpu.async_copy(ad_hbm.at[pair_i.at[q, 1]], ade_v.at[b],
                         sem_ad.at[b])

    def wait_gather(j):
        b, q = j % 3, j % 4
        pltpu.make_async_copy(theta_hbm.at[pair_i.at[q, 0]], msg_v.at[b],
                              sem_rows.at[b]).wait()
        pltpu.make_async_copy(as_hbm.at[pair_i.at[q, 0]], ase_v.at[b],
                              sem_as.at[b]).wait()
        pltpu.make_async_copy(ad_hbm.at[pair_i.at[q, 1]], ade_v.at[b],
                              sem_ad.at[b]).wait()

    def start_scatter(j):
        b, q = j % 3, j % 4
        pltpu.async_copy(msg_v.at[b], acc_sp.at[pair_i.at[q, 1]],
                         sem_sc.at[b], add=True)

    def wait_scatter(j):
        b, q = j % 3, j % 4
        pltpu.make_async_copy(msg_v.at[b], acc_sp.at[pair_i.at[q, 1]],
                              sem_sc.at[b]).wait()

    start_idx(0)
    start_idx(1)
    wait_idx(0)
    start_gather(0)

    def compute_and_scatter(j):
        b, q = j % 3, j % 4
        wait_gather(j)
        for k in range(CH // 16):
            sl = pl.ds(k * 16, 16)
            a = ase_v[b, sl] + ade_v[b, sl]
            e = jnp.where(a >= 0.0, a, 0.2 * a)
            w16 = jnp.exp(e)
            w_v[sl] = w16
            d16 = pair_i[q, 1, sl]
            plsc.addupdate_scatter(
                den_v, [lax.shift_right_logical(d16, 7),
                        lax.bitwise_and(d16, 127)], w16)
        for ei in range(CH):
            wsp = plsc.load_gather(w_v, [rep_v[pl.ds(ei * 16, 16)]])
            for k in range(D // 16):
                sl = pl.ds(k * 16, 16)
                msg_v[b, ei, sl] = msg_v[b, ei, sl] * wsp
        start_scatter(j)

    def chunk_guarded(j, carry):
        @pl.when(j >= 2)
        def _():
            wait_scatter(j - 2)

        @pl.when(j + 2 < NCHUNK)
        def _():
            start_idx(j + 2)

        @pl.when(j + 1 < NCHUNK)
        def _():
            wait_idx(j + 1)
            start_gather(j + 1)

        compute_and_scatter(j)
        return carry

    lax.fori_loop(0, NCHUNK, chunk_guarded, 0)
    wait_scatter(NCHUNK - 2)
    wait_scatter(NCHUNK - 1)
    plsc.subcore_barrier()
    pltpu.sync_copy(den_v, den_sp.at[r80_v], add=True)
    plsc.subcore_barrier()
    pltpu.sync_copy(acc_sp.at[pl.ds(s * RPT, RPT)],
                    num_hbm.at[c, pl.ds(s * RPT, RPT)])
    pltpu.sync_copy(den_sp.at[pl.ds(s * 5, 5)],
                    den_hbm.at[c, pl.ds(s * 5, 5)])


@functools.partial(
    pl.kernel,
    out_type=[jax.ShapeDtypeStruct((NC, N, D), jnp.float32),
              jax.ShapeDtypeStruct((NC, 80, 128), jnp.float32)],
    mesh=plsc.VectorSubcoreMesh(core_axis_name="c", subcore_axis_name="s"),
    compiler_params=pltpu.CompilerParams(use_tc_tiling_on_sc=False,
                                         needs_layout_passes=False),
    scratch_types=[
        pltpu.VMEM_SHARED((N, D), jnp.float32),
        pltpu.VMEM_SHARED((80, 128), jnp.float32),
        pltpu.VMEM((4, 2, CH), jnp.int32),
        pltpu.VMEM((3, CH), jnp.float32),
        pltpu.VMEM((3, CH), jnp.float32),
        pltpu.VMEM((CH,), jnp.float32),
        pltpu.VMEM((3, CH, D), jnp.float32),
        pltpu.VMEM((80, 128), jnp.float32),
        pltpu.VMEM((CH * 16,), jnp.int32),
        pltpu.VMEM((80,), jnp.int32),
        pltpu.SemaphoreType.DMA((3,)),
        pltpu.SemaphoreType.DMA((3,)),
        pltpu.SemaphoreType.DMA((3,)),
        pltpu.SemaphoreType.DMA((4,)),
        pltpu.SemaphoreType.DMA((3,)),
    ],
)
def _sc_edge(theta_hbm, as_hbm, ad_hbm, pair_hbm, zeros_hbm, rep_hbm,
             r80_hbm, num_hbm, den_hbm, acc_sp, den_sp, pair_i, ase_v,
             ade_v, w_v, msg_v, den_v, rep_v, r80_v, sem_rows, sem_as,
             sem_ad, sem_idx, sem_sc):
    _sc_edge_body(theta_hbm, as_hbm, ad_hbm, pair_hbm, zeros_hbm, rep_hbm,
                  r80_hbm, num_hbm, den_hbm, acc_sp, den_sp, pair_i,
                  ase_v, ade_v, w_v, msg_v, den_v, rep_v, r80_v,
                  sem_rows, sem_as, sem_ad, sem_idx, sem_sc)


def kernel(x, edge_index, W1, att_src1, att_dst1, b1,
           W2, att_src2, att_dst2, b2):
    ei = edge_index.astype(jnp.int32)
    pair = jnp.stack([ei[0].reshape(NW, NCHUNK, CH),
                      ei[1].reshape(NW, NCHUNK, CH)], axis=2)
    zeros = jnp.zeros((N, D), jnp.float32)
    rep = jnp.repeat(jnp.arange(CH, dtype=jnp.int32), 16)
    r80 = jnp.arange(80, dtype=jnp.int32)

    th1, as1, ad1 = _tc_transform(x, W1, att_src1.reshape(1, D),
                                  att_dst1.reshape(1, D))
    num1, den1f = _sc_edge(th1, as1.reshape(N), ad1.reshape(N),
                           pair, zeros, rep, r80)
    th2, as2, ad2 = _tc_mid(num1, den1f.reshape(NC, 80 * 128, 1),
                            b1.reshape(1, D), W2, att_src2.reshape(1, D),
                            att_dst2.reshape(1, D))
    num2, den2f = _sc_edge(th2, as2.reshape(N), ad2.reshape(N),
                           pair, zeros, rep, r80)
    return _tc_out(num2, den2f.reshape(NC, 80 * 128, 1),
                   b2.reshape(1, D))
